# baseline (speedup 1.0000x reference)
import functools

import jax
import jax.numpy as jnp
from jax import lax
from jax.experimental import pallas as pl
from jax.experimental.pallas import tpu as pltpu
from jax.experimental.pallas import tpu_sc as plsc

_info = plsc.get_sparse_core_info()
_NC = _info.num_cores
_NS = _info.num_subcores
_NW = _NC * _NS


def _make_gather(t: int, b: int, d: int):
    cols = b // _NW
    ng = cols // 16

    mesh = plsc.VectorSubcoreMesh(core_axis_name="c", subcore_axis_name="s")

    @functools.partial(
        pl.kernel,
        mesh=mesh,
        compiler_params=pltpu.CompilerParams(
            use_tc_tiling_on_sc=False, needs_layout_passes=False),
        out_type=jax.ShapeDtypeStruct((t, b, d), jnp.float32),
        scratch_types=[
            pltpu.VMEM((t, cols), jnp.int32),
            pltpu.VMEM((cols, 2 * d), jnp.float32),
            pltpu.VMEM((cols, 2 * d), jnp.float32),
            pltpu.VMEM((cols, d), jnp.float32),
            pltpu.VMEM((cols, d), jnp.float32),
            pltpu.SemaphoreType.DMA,
            pltpu.SemaphoreType.DMA,
            pltpu.SemaphoreType.DMA,
        ],
    )
    def gather_kernel(idx_hbm, table_hbm, out_hbm,
                      idx_v, st0, st1, ob0, ob1, gsem0, gsem1, ssem):
        wid = lax.axis_index("s") * _NC + lax.axis_index("c")
        col0 = wid * cols
        pltpu.sync_copy(idx_hbm.at[:, pl.ds(col0, cols)], idx_v)

        staged = (st0, st1)
        obufs = (ob0, ob1)
        gsems = (gsem0, gsem1)
        iota16 = lax.iota(jnp.int32, 16)

        def fire(row, sbuf, gsem):
            for u in range(ng):
                iv = idx_v[row, pl.ds(u * 16, 16)]
                slab = lax.shift_right_logical(iv, 1)
                pltpu.async_copy(
                    table_hbm.at[slab], sbuf.at[pl.ds(u * 16, 16)], gsem)

        def drain(row, sbuf, gsem):
            for u in range(ng):
                iv = idx_v[row, pl.ds(u * 16, 16)]
                slab = lax.shift_right_logical(iv, 1)
                pltpu.make_async_copy(
                    table_hbm.at[slab], sbuf.at[pl.ds(u * 16, 16)],
                    gsem).wait()

        def extract(row, sbuf, obuf):
            def ubody(u, carry):
                iv = idx_v[row, pl.ds(u * 16, 16)]
                rows16 = u * 16 + iota16
                coloff = (iv & 1) * d
                for dd in range(d):
                    vals = plsc.load_gather(sbuf, [rows16, coloff + dd])
                    plsc.store_scatter(
                        obuf, [rows16, jnp.full((16,), dd, jnp.int32)], vals)
                return carry
            lax.fori_loop(0, ng, ubody, 0)

        def store_slice(row):
            return out_hbm.at[row, pl.ds(col0, cols)]

        def drain_store(row, obuf):
            pltpu.make_async_copy(store_slice(row), obuf, ssem).wait()

        fire(0, staged[0], gsems[0])

        def body(i, carry):
            t0 = 2 * i
            t1 = t0 + 1
            fire(t1, staged[1], gsems[1])
            drain(t0, staged[0], gsems[0])

            @pl.when(i > 0)
            def _():
                drain_store(t0, obufs[0])
            extract(t0, staged[0], obufs[0])
            pltpu.async_copy(obufs[0], store_slice(t0), ssem)

            @pl.when(i < (t // 2) - 1)
            def _():
                fire(t0 + 2, staged[0], gsems[0])
            drain(t1, staged[1], gsems[1])

            @pl.when(i > 0)
            def _():
                drain_store(t1, obufs[1])
            extract(t1, staged[1], obufs[1])
            pltpu.async_copy(obufs[1], store_slice(t1), ssem)
            return carry

        lax.fori_loop(0, t // 2, body, 0)
        drain_store(t - 2, obufs[0])
        drain_store(t - 1, obufs[1])

    return gather_kernel


def kernel(sentences, table):
    t, b = sentences.shape
    v, d = table.shape
    tbl2 = table.reshape(v // 2, 2 * d)
    out = _make_gather(t, b, d)(sentences.astype(jnp.int32), tbl2)
    return out


# speedup vs baseline: 1.5586x; 1.5586x over previous
import functools

import jax
import jax.numpy as jnp
from jax import lax
from jax.experimental import pallas as pl
from jax.experimental.pallas import tpu as pltpu
from jax.experimental.pallas import tpu_sc as plsc

_info = plsc.get_sparse_core_info()
_NC = _info.num_cores
_NS = _info.num_subcores
_NW = _NC * _NS

_K = 5


def _make_gather(t: int, b: int, d: int):
    cols = b // _NW
    n_chunks = t // _K
    spt = cols // 16

    mesh = plsc.VectorSubcoreMesh(core_axis_name="c", subcore_axis_name="s")

    @functools.partial(
        pl.kernel,
        mesh=mesh,
        compiler_params=pltpu.CompilerParams(
            use_tc_tiling_on_sc=False, needs_layout_passes=False),
        out_type=jax.ShapeDtypeStruct((t, b, d), jnp.float32),
        scratch_types=[
            pltpu.VMEM((t, cols), jnp.int32),
            pltpu.VMEM((_K, cols, d), jnp.float32),
            pltpu.VMEM((_K, cols, d), jnp.float32),
            pltpu.SemaphoreType.DMA,
            pltpu.SemaphoreType.DMA,
            pltpu.SemaphoreType.DMA,
        ],
    )
    def gather_kernel(idx_hbm, table_hbm, out_hbm,
                      idx_v, buf0, buf1, gsem0, gsem1, ssem):
        wid = lax.axis_index("s") * _NC + lax.axis_index("c")
        col0 = wid * cols
        pltpu.sync_copy(idx_hbm.at[:, pl.ds(col0, cols)], idx_v)

        bufs = (buf0, buf1)
        gsems = (gsem0, gsem1)

        def fire(chunk, bi):
            cps = []
            for j in range(_K):
                for u in range(spt):
                    iv = idx_v[chunk * _K + j, pl.ds(u * 16, 16)]
                    cps.append(pltpu.async_copy(
                        table_hbm.at[iv],
                        bufs[bi].at[j, pl.ds(u * 16, 16)],
                        gsems[bi]))
            return cps

        pending_store = [None, None]
        gathers = [None, None]
        gathers[0] = fire(0, 0)
        for c in range(n_chunks):
            bi = c % 2
            ni = (c + 1) % 2
            if c + 1 < n_chunks:
                if pending_store[ni] is not None:
                    pending_store[ni].wait()
                    pending_store[ni] = None
                gathers[ni] = fire(c + 1, ni)
            for cp in gathers[bi]:
                cp.wait()
            pending_store[bi] = pltpu.async_copy(
                bufs[bi],
                out_hbm.at[pl.ds(c * _K, _K), pl.ds(col0, cols)],
                ssem)
        for st in pending_store:
            if st is not None:
                st.wait()

    return gather_kernel


def kernel(sentences, table):
    t, b = sentences.shape
    v, d = table.shape
    tbl = lax.optimization_barrier(table)
    out = _make_gather(t, b, d)(sentences.astype(jnp.int32), tbl)
    return out


# speedup vs baseline: 1.5606x; 1.0012x over previous
import functools

import jax
import jax.numpy as jnp
from jax import lax
from jax.experimental import pallas as pl
from jax.experimental.pallas import tpu as pltpu
from jax.experimental.pallas import tpu_sc as plsc

_info = plsc.get_sparse_core_info()
_NC = _info.num_cores
_NS = _info.num_subcores
_NW = _NC * _NS

_K = 5


def _make_gather(t: int, b: int, d: int):
    cols = b // _NW
    n_chunks = t // _K
    spt = cols // 16

    mesh = plsc.VectorSubcoreMesh(core_axis_name="c", subcore_axis_name="s")

    @functools.partial(
        pl.kernel,
        mesh=mesh,
        compiler_params=pltpu.CompilerParams(
            use_tc_tiling_on_sc=False, needs_layout_passes=False),
        out_type=jax.ShapeDtypeStruct((t, b, d), jnp.float32),
        scratch_types=[
            pltpu.VMEM((t, cols), jnp.int32),
            pltpu.VMEM((_K, cols, d), jnp.float32),
            pltpu.VMEM((_K, cols, d), jnp.float32),
            pltpu.SemaphoreType.DMA,
            pltpu.SemaphoreType.DMA,
            pltpu.SemaphoreType.DMA,
        ],
    )
    def gather_kernel(idx_hbm, table_hbm, out_hbm,
                      idx_v, buf0, buf1, gsem0, gsem1, ssem):
        wid = lax.axis_index("s") * _NC + lax.axis_index("c")
        col0 = wid * cols
        pltpu.sync_copy(idx_hbm.at[:, pl.ds(col0, cols)], idx_v)

        bufs = (buf0, buf1)
        gsems = (gsem0, gsem1)

        def fire(chunk, bi):
            cps = []
            for j in range(_K):
                for u in range(spt):
                    iv = idx_v[chunk * _K + j, pl.ds(u * 16, 16)]
                    cps.append(pltpu.async_copy(
                        table_hbm.at[iv],
                        bufs[bi].at[j, pl.ds(u * 16, 16)],
                        gsems[bi]))
            return cps

        pending_store = [None, None]
        gathers = [None, None]
        gathers[0] = fire(0, 0)
        for c in range(n_chunks):
            bi = c % 2
            ni = (c + 1) % 2
            if c + 1 < n_chunks:
                if pending_store[ni] is not None:
                    pending_store[ni].wait()
                    pending_store[ni] = None
                gathers[ni] = fire(c + 1, ni)
            for cp in gathers[bi]:
                cp.wait()
            pending_store[bi] = pltpu.async_copy(
                bufs[bi],
                out_hbm.at[pl.ds(c * _K, _K), pl.ds(col0, cols)],
                ssem)
        for st in pending_store:
            if st is not None:
                st.wait()

    return gather_kernel


def kernel(sentences, table):
    t, b = sentences.shape
    v, d = table.shape
    out = _make_gather(t, b, d)(sentences.astype(jnp.int32), table)
    return out
